# Initial kernel scaffold; baseline (speedup 1.0000x reference)
import functools

import jax
import jax.numpy as jnp
from jax import lax
from jax.experimental import pallas as pl
from jax.experimental.pallas import tpu as pltpu
from jax.experimental.pallas import tpu_sc as plsc

_NC = 2
_NS = 16
_NW = _NC * _NS


def _make_gather(total, embed_dim, chunk):
  n_chunks_total = total // chunk
  mesh = plsc.VectorSubcoreMesh(core_axis_name="c", subcore_axis_name="s")

  @functools.partial(
      pl.kernel,
      mesh=mesh,
      out_type=jax.ShapeDtypeStruct((total, embed_dim), jnp.float32),
      scratch_types=[
          pltpu.VMEM((chunk,), jnp.int32),
          pltpu.VMEM((chunk, embed_dim), jnp.float32),
          pltpu.SemaphoreType.DMA,
      ],
  )
  def gather_kernel(idx_hbm, table_hbm, out_hbm, idx_v, rows_v, sem):
    wid = lax.axis_index("s") * _NC + lax.axis_index("c")
    n_per_w = n_chunks_total // _NW

    def body(g, carry):
      off = (wid * n_per_w + g) * chunk
      pltpu.sync_copy(idx_hbm.at[pl.ds(off, chunk)], idx_v)
      pltpu.async_copy(table_hbm.at[idx_v], rows_v, sem).wait()
      pltpu.sync_copy(rows_v, out_hbm.at[pl.ds(off, chunk)])
      return carry

    lax.fori_loop(0, n_per_w, body, 0)

  return gather_kernel


def kernel(x, table):
  batch, timesteps = x.shape
  vocab, embed_dim = table.shape
  total = batch * timesteps
  chunk = 1024
  assert total % (_NW * chunk) == 0
  flat_idx = x.reshape(total).astype(jnp.int32)
  out = _make_gather(total, embed_dim, chunk)(flat_idx, table)
  return out.reshape(batch, timesteps, embed_dim)


# speedup vs baseline: 1.4594x; 1.4594x over previous
import functools

import jax
import jax.numpy as jnp
from jax import lax
from jax.experimental import pallas as pl
from jax.experimental.pallas import tpu as pltpu
from jax.experimental.pallas import tpu_sc as plsc

_NC = 2
_NS = 16
_NW = _NC * _NS


def _make_gather(total, embed_dim, chunk):
  n_chunks_total = total // chunk
  mesh = plsc.VectorSubcoreMesh(core_axis_name="c", subcore_axis_name="s")

  @functools.partial(
      pl.kernel,
      mesh=mesh,
      out_type=jax.ShapeDtypeStruct((total, embed_dim), jnp.float32),
      scratch_types=[
          pltpu.VMEM((chunk,), jnp.int32),
          pltpu.VMEM((chunk, embed_dim), jnp.float32),
          pltpu.SemaphoreType.DMA,
      ],
      compiler_params=pltpu.CompilerParams(use_tc_tiling_on_sc=False),
  )
  def gather_kernel(idx_hbm, table_hbm, out_hbm, idx_v, rows_v, sem):
    wid = lax.axis_index("s") * _NC + lax.axis_index("c")
    n_per_w = n_chunks_total // _NW

    def body(g, carry):
      off = (wid * n_per_w + g) * chunk
      pltpu.sync_copy(idx_hbm.at[pl.ds(off, chunk)], idx_v)
      pltpu.async_copy(table_hbm.at[idx_v], rows_v, sem).wait()
      pltpu.sync_copy(rows_v, out_hbm.at[pl.ds(off, chunk)])
      return carry

    lax.fori_loop(0, n_per_w, body, 0)

  return gather_kernel


def kernel(x, table):
  batch, timesteps = x.shape
  vocab, embed_dim = table.shape
  total = batch * timesteps
  chunk = 1024
  assert total % (_NW * chunk) == 0
  flat_idx = x.reshape(total).astype(jnp.int32)
  out = _make_gather(total, embed_dim, chunk)(flat_idx, table)
  return out.reshape(batch, timesteps, embed_dim)


# speedup vs baseline: 1.5022x; 1.0293x over previous
import functools

import jax
import jax.numpy as jnp
from jax import lax
from jax.experimental import pallas as pl
from jax.experimental.pallas import tpu as pltpu
from jax.experimental.pallas import tpu_sc as plsc

_NC = 2
_NS = 16
_NW = _NC * _NS


def _make_gather(total, embed_dim, chunk):
  b_per_w = total // _NW
  n_chunks = b_per_w // chunk
  assert n_chunks % 2 == 0
  mesh = plsc.VectorSubcoreMesh(core_axis_name="c", subcore_axis_name="s")

  @functools.partial(
      pl.kernel,
      mesh=mesh,
      out_type=jax.ShapeDtypeStruct((total, embed_dim), jnp.float32),
      scratch_types=[
          pltpu.VMEM((n_chunks, chunk), jnp.int32),
          pltpu.VMEM((chunk, embed_dim), jnp.float32),
          pltpu.VMEM((chunk, embed_dim), jnp.float32),
          pltpu.SemaphoreType.DMA,
          pltpu.SemaphoreType.DMA,
          pltpu.SemaphoreType.DMA,
          pltpu.SemaphoreType.DMA,
      ],
      compiler_params=pltpu.CompilerParams(use_tc_tiling_on_sc=False),
  )
  def gather_kernel(idx_hbm, table_hbm, out_hbm, idx_v, rows0, rows1,
                    sem_g0, sem_g1, sem_o0, sem_o1):
    wid = lax.axis_index("s") * _NC + lax.axis_index("c")
    base = wid * b_per_w
    rows = (rows0, rows1)
    sem_g = (sem_g0, sem_g1)
    sem_o = (sem_o0, sem_o1)

    pltpu.sync_copy(idx_hbm.at[wid], idx_v)

    def gather_start(g, b):
      pltpu.async_copy(table_hbm.at[idx_v.at[g]], rows[b], sem_g[b])

    def gather_wait(g, b):
      pltpu.make_async_copy(table_hbm.at[idx_v.at[g]], rows[b],
                            sem_g[b]).wait()

    def out_start(g, b):
      pltpu.async_copy(rows[b], out_hbm.at[pl.ds(base + g * chunk, chunk)],
                       sem_o[b])

    def out_wait(g, b):
      pltpu.make_async_copy(rows[b],
                            out_hbm.at[pl.ds(base + g * chunk, chunk)],
                            sem_o[b]).wait()

    gather_start(0, 0)

    def step(g, b):
      @pl.when(g + 1 < n_chunks)
      def _():
        @pl.when(g >= 1)
        def _():
          out_wait(g - 1, 1 - b)

        gather_start(g + 1, 1 - b)

      gather_wait(g, b)
      out_start(g, b)

    def body(j, carry):
      step(2 * j, 0)
      step(2 * j + 1, 1)
      return carry

    lax.fori_loop(0, n_chunks // 2, body, 0)
    out_wait(n_chunks - 2, 0)
    out_wait(n_chunks - 1, 1)

  return gather_kernel


def kernel(x, table):
  batch, timesteps = x.shape
  vocab, embed_dim = table.shape
  total = batch * timesteps
  chunk = 1280
  assert total % (_NW * chunk) == 0
  idx = x.reshape(_NW, total // (_NW * chunk), chunk).astype(jnp.int32)
  out = _make_gather(total, embed_dim, chunk)(idx, table)
  return out.reshape(batch, timesteps, embed_dim)
